# baseline (speedup 1.0000x reference)
import functools

import jax
import jax.numpy as jnp
from jax import lax
from jax.experimental import pallas as pl
from jax.experimental.pallas import tpu as pltpu
from jax.experimental.pallas import tpu_sc as plsc

_B, _S, _D = 4, 4096, 2048
_T = 3
_NW = 32
_WPB = _NW // _B
_RPW = _S // _WPB
_CH = 16
_NI = _RPW // _CH


def _sc_concat(x_hbm, emb_hbm, out_hbm, buf0, buf1, ebuf, sem_r, sem_w):
    c = lax.axis_index("c")
    s = lax.axis_index("s")
    wid = s * 2 + c
    b = wid // _WPB
    wk = wid % _WPB
    r0 = wk * _RPW
    bufs = (buf0, buf1)

    @pl.when(wk == 0)
    def _():
        pltpu.sync_copy(emb_hbm, ebuf.at[pl.ds(0, _T)])
        pltpu.sync_copy(ebuf.at[pl.ds(0, _T)], out_hbm.at[b, pl.ds(0, _T)])

    def gather(i, buf):
        pltpu.async_copy(x_hbm.at[b, pl.ds(r0 + _CH * i, _CH)], buf, sem_r)

    def write(i, buf):
        pltpu.async_copy(buf, out_hbm.at[b, pl.ds(r0 + _CH * i + _T, _CH)],
                         sem_w)

    def wait_one(sem):
        pltpu.make_async_copy(x_hbm.at[b, pl.ds(0, _CH)], buf0, sem).wait()

    gather(0, bufs[0])

    def body(i2, carry):
        for j in range(2):
            i = i2 * 2 + j
            cur = bufs[j]
            nxt = bufs[1 - j]
            wait_one(sem_r)
            write(i, cur)

            @pl.when(i >= 1)
            def _():
                wait_one(sem_w)

            @pl.when(i + 1 < _NI)
            def _():
                gather(i + 1, nxt)
        return carry

    lax.fori_loop(0, _NI // 2, body, 0)
    wait_one(sem_w)


def kernel(x, embedding):
    mesh = plsc.VectorSubcoreMesh(core_axis_name="c", subcore_axis_name="s")
    k = functools.partial(
        pl.kernel,
        mesh=mesh,
        out_type=jax.ShapeDtypeStruct((_B, _S + _T, _D), x.dtype),
        compiler_params=pltpu.CompilerParams(use_tc_tiling_on_sc=False),
        scratch_types=[
            pltpu.VMEM((_CH, _D), jnp.float32),
            pltpu.VMEM((_CH, _D), jnp.float32),
            pltpu.VMEM((_CH, _D), jnp.float32),
            pltpu.SemaphoreType.DMA,
            pltpu.SemaphoreType.DMA,
        ],
    )(_sc_concat)
    return k(x, embedding)


# speedup vs baseline: 3.3735x; 3.3735x over previous
import functools

import jax
import jax.numpy as jnp
from jax import lax
from jax.experimental import pallas as pl
from jax.experimental.pallas import tpu as pltpu
from jax.experimental.pallas import tpu_sc as plsc

_B, _S, _D = 4, 4096, 2048
_T = 3
_NW = 32
_WPB = _NW // _B
_RPW = _S // _WPB
_CH = 16
_NI = _RPW // _CH


def _sc_concat(x_hbm, emb_hbm, out_hbm, buf0, buf1, ebuf, sem_r, sem_w):
    c = lax.axis_index("c")
    s = lax.axis_index("s")
    wid = s * 2 + c
    b = wid // _WPB
    wk = wid % _WPB
    r0 = wk * _RPW
    bufs = (buf0, buf1)
    lanes = lax.iota(jnp.int32, _CH)
    is_a = wk < 4

    @pl.when(wk == 0)
    def _():
        pltpu.sync_copy(emb_hbm, ebuf.at[pl.ds(0, _T)])
        pltpu.sync_copy(ebuf.at[pl.ds(0, _T)], out_hbm.at[b, pl.ds(0, _T)])

    @pl.when(wk >= 4)
    def _():
        pltpu.sync_copy(x_hbm.at[b, pl.ds(r0, _CH)], ebuf)
        pltpu.async_copy(ebuf, out_hbm.at[b].at[(r0 + _T) + lanes], sem_w)
        pltpu.make_async_copy(x_hbm.at[b, pl.ds(0, _CH)], ebuf, sem_w).wait()

    @pl.when(wk == _WPB - 1)
    def _():
        pltpu.sync_copy(x_hbm.at[b, pl.ds(_S - _CH, _CH)], ebuf)
        pltpu.async_copy(ebuf, out_hbm.at[b].at[(_S - _CH + _T) + lanes],
                         sem_w)
        pltpu.make_async_copy(x_hbm.at[b, pl.ds(0, _CH)], ebuf, sem_w).wait()

    def valid(i):
        return is_a | (wk < _WPB - 1) | (i < _NI - 1)

    def gather(i, buf):
        @pl.when(is_a)
        def _():
            pltpu.async_copy(x_hbm.at[b, pl.ds(r0 + _CH * i, _CH)], buf,
                             sem_r)

        @pl.when(jnp.logical_not(is_a))
        def _():
            pltpu.async_copy(x_hbm.at[b].at[(r0 + 5 + _CH * i) + lanes], buf,
                             sem_r)

    def write(i, buf):
        @pl.when(is_a)
        def _():
            pltpu.async_copy(buf, out_hbm.at[b].at[(r0 + _T + _CH * i) + lanes],
                             sem_w)

        @pl.when(jnp.logical_not(is_a))
        def _():
            pltpu.async_copy(buf, out_hbm.at[b, pl.ds(r0 + 8 + _CH * i, _CH)],
                             sem_w)

    def wait_one(sem):
        pltpu.make_async_copy(x_hbm.at[b, pl.ds(0, _CH)], buf0, sem).wait()

    gather(0, bufs[0])

    def body(i2, carry):
        for j in range(2):
            i = i2 * 2 + j
            cur = bufs[j]
            nxt = bufs[1 - j]

            @pl.when(valid(i))
            def _():
                wait_one(sem_r)
                write(i, cur)

            @pl.when((i >= 1) & valid(i))
            def _():
                wait_one(sem_w)

            @pl.when(valid(i + 1))
            def _():
                gather(i + 1, nxt)
        return carry

    lax.fori_loop(0, _NI // 2, body, 0)
    wait_one(sem_w)


def kernel(x, embedding):
    mesh = plsc.VectorSubcoreMesh(core_axis_name="c", subcore_axis_name="s")
    k = functools.partial(
        pl.kernel,
        mesh=mesh,
        out_type=jax.ShapeDtypeStruct((_B, _S + _T, _D), x.dtype),
        scratch_types=[
            pltpu.VMEM((_CH, _D), jnp.float32),
            pltpu.VMEM((_CH, _D), jnp.float32),
            pltpu.VMEM((_CH, _D), jnp.float32),
            pltpu.SemaphoreType.DMA,
            pltpu.SemaphoreType.DMA,
        ],
    )(_sc_concat)
    return k(x, embedding)
